# Initial kernel scaffold; baseline (speedup 1.0000x reference)
import jax
import jax.numpy as jnp
from jax.experimental import pallas as pl


def kernel(agent_hist, lane_nodes, edge_index_aa, edge_index_al, W_ih_a, W_hh_a, b_ih_a, b_hh_a, W_ih_l, W_hh_l, b_ih_l, b_hh_l, W1, as1, ad1, b1, bn_g, bn_b, bn_m, bn_v, W2, as2, ad2, b2):
    raise NotImplementedError("write your pallas kernel here")


# speedup vs baseline: 42.3775x; 42.3775x over previous
import functools

import numpy as np
import jax
import jax.numpy as jnp
from jax import lax
from jax.experimental import pallas as pl
from jax.experimental.pallas import tpu as pltpu
from jax.experimental.pallas import tpu_sc as plsc

A_TOT, T_H, AG_IN = 10000, 20, 5
L_TOT, P_LEN, LN_IN = 40000, 10, 2
HID = 64
HEADS, HDIM = 4, 16
E_AA, E_AL = 400000, 200000
N_TOT = A_TOT + L_TOT
E_TOT = E_AA + 2 * E_AL

NC, NS, LANES = 2, 16, 16
NW = NC * NS

_g16 = np.zeros((HID, 16), np.float32)
for _h in range(HEADS):
    _g16[_h * HDIM:(_h + 1) * HDIM, _h] = 1.0
_e4 = np.zeros((HEADS, HID), np.float32)
for _h in range(HEADS):
    _e4[_h, _h * HDIM:(_h + 1) * HDIM] = 1.0


def _lstm_body(x_ref, wih_ref, whh_ref, b_ref, out_ref, *, T, mean):
    wih = wih_ref[...]
    whh = whh_ref[...]
    b = b_ref[...]
    B = x_ref.shape[1]
    h0 = jnp.zeros((B, HID), jnp.float32)

    def step(t, carry):
        h, c, acc = carry
        xt = x_ref[t]
        g = (jnp.dot(xt, wih, preferred_element_type=jnp.float32)
             + jnp.dot(h, whh, preferred_element_type=jnp.float32) + b)
        i = jax.nn.sigmoid(g[:, 0:HID])
        f = jax.nn.sigmoid(g[:, HID:2 * HID])
        gg = jnp.tanh(g[:, 2 * HID:3 * HID])
        o = jax.nn.sigmoid(g[:, 3 * HID:4 * HID])
        c2 = f * c + i * gg
        h2 = o * jnp.tanh(c2)
        return h2, c2, acc + h2

    hT, _, hacc = lax.fori_loop(0, T, step, (h0, h0, h0))
    out_ref[...] = (hacc / T) if mean else hT


def _lstm_encode(x_t, wihT, whhT, b2d, T, F, mean, blk):
    n = x_t.shape[1]
    return pl.pallas_call(
        functools.partial(_lstm_body, T=T, mean=mean),
        grid=(n // blk,),
        in_specs=[
            pl.BlockSpec((T, blk, F), lambda i: (0, i, 0)),
            pl.BlockSpec((F, 4 * HID), lambda i: (0, 0)),
            pl.BlockSpec((HID, 4 * HID), lambda i: (0, 0)),
            pl.BlockSpec((1, 4 * HID), lambda i: (0, 0)),
        ],
        out_specs=pl.BlockSpec((blk, HID), lambda i: (i, 0)),
        out_shape=jax.ShapeDtypeStruct((n, HID), jnp.float32),
    )(x_t, wihT, whhT, b2d)


def _prep_body(x_ref, wT_ref, af_ref, adf_ref, g16_ref, xw_ref, sds_ref, sdd_ref):
    xw = jnp.dot(x_ref[...], wT_ref[...], preferred_element_type=jnp.float32)
    g16 = g16_ref[...]
    xw_ref[...] = xw
    sds_ref[...] = jnp.dot(xw * af_ref[...], g16, preferred_element_type=jnp.float32)
    sdd_ref[...] = jnp.dot(xw * adf_ref[...], g16, preferred_element_type=jnp.float32)


def _prep(x, wT, af, adf, g16, blk=1000):
    n = x.shape[0]
    return pl.pallas_call(
        _prep_body,
        grid=(n // blk,),
        in_specs=[
            pl.BlockSpec((blk, HID), lambda i: (i, 0)),
            pl.BlockSpec((HID, HID), lambda i: (0, 0)),
            pl.BlockSpec((1, HID), lambda i: (0, 0)),
            pl.BlockSpec((1, HID), lambda i: (0, 0)),
            pl.BlockSpec((HID, 16), lambda i: (0, 0)),
        ],
        out_specs=[
            pl.BlockSpec((blk, HID), lambda i: (i, 0)),
            pl.BlockSpec((blk, 16), lambda i: (i, 0)),
            pl.BlockSpec((blk, 16), lambda i: (i, 0)),
        ],
        out_shape=[
            jax.ShapeDtypeStruct((n, HID), jnp.float32),
            jax.ShapeDtypeStruct((n, 16), jnp.float32),
            jax.ShapeDtypeStruct((n, 16), jnp.float32),
        ],
    )(x, wT, af, adf, g16)


def _mid_body(acc_ref, denp_ref, x_ref, sc_ref, cst_ref, e4_ref,
              w2T_ref, af2_ref, adf2_ref, g16_ref,
              h_ref, xw2_ref, sds2_ref, sdd2_ref):
    dp = denp_ref[...]
    den = dp[0, :, 0:HEADS] + dp[1, :, 0:HEADS]
    inv = 1.0 / (den + 1e-16)
    inv64 = jnp.dot(inv, e4_ref[...], preferred_element_type=jnp.float32)
    y = acc_ref[...] * inv64 * sc_ref[...] + cst_ref[...]
    el = jnp.where(y > 0, y, jnp.exp(jnp.minimum(y, 0.0)) - 1.0)
    h = el + x_ref[...]
    h_ref[...] = h
    xw2 = jnp.dot(h, w2T_ref[...], preferred_element_type=jnp.float32)
    g16 = g16_ref[...]
    xw2_ref[...] = xw2
    sds2_ref[...] = jnp.dot(xw2 * af2_ref[...], g16, preferred_element_type=jnp.float32)
    sdd2_ref[...] = jnp.dot(xw2 * adf2_ref[...], g16, preferred_element_type=jnp.float32)


def _mid(acc64, denp, x, sc2d, cst2d, e4, w2T, af2, adf2, g16, blk=1000):
    n = x.shape[0]
    return pl.pallas_call(
        _mid_body,
        grid=(n // blk,),
        in_specs=[
            pl.BlockSpec((blk, HID), lambda i: (i, 0)),
            pl.BlockSpec((2, blk, 16), lambda i: (0, i, 0)),
            pl.BlockSpec((blk, HID), lambda i: (i, 0)),
            pl.BlockSpec((1, HID), lambda i: (0, 0)),
            pl.BlockSpec((1, HID), lambda i: (0, 0)),
            pl.BlockSpec((HEADS, HID), lambda i: (0, 0)),
            pl.BlockSpec((HID, HID), lambda i: (0, 0)),
            pl.BlockSpec((1, HID), lambda i: (0, 0)),
            pl.BlockSpec((1, HID), lambda i: (0, 0)),
            pl.BlockSpec((HID, 16), lambda i: (0, 0)),
        ],
        out_specs=[
            pl.BlockSpec((blk, HID), lambda i: (i, 0)),
            pl.BlockSpec((blk, HID), lambda i: (i, 0)),
            pl.BlockSpec((blk, 16), lambda i: (i, 0)),
            pl.BlockSpec((blk, 16), lambda i: (i, 0)),
        ],
        out_shape=[
            jax.ShapeDtypeStruct((n, HID), jnp.float32),
            jax.ShapeDtypeStruct((n, HID), jnp.float32),
            jax.ShapeDtypeStruct((n, 16), jnp.float32),
            jax.ShapeDtypeStruct((n, 16), jnp.float32),
        ],
    )(acc64, denp, x, sc2d, cst2d, e4, w2T, af2, adf2, g16)


def _final_body(acc_ref, denp_ref, h_ref, b2_ref, e4_ref, out_ref):
    dp = denp_ref[...]
    den = dp[0, :, 0:HEADS] + dp[1, :, 0:HEADS]
    inv = 1.0 / (den + 1e-16)
    inv64 = jnp.dot(inv, e4_ref[...], preferred_element_type=jnp.float32)
    out_ref[...] = acc_ref[...] * inv64 + b2_ref[...] + h_ref[...]


def _final(acc64, denp, h, b2d, e4, blk=1000):
    n = h.shape[0]
    return pl.pallas_call(
        _final_body,
        grid=(n // blk,),
        in_specs=[
            pl.BlockSpec((blk, HID), lambda i: (i, 0)),
            pl.BlockSpec((2, blk, 16), lambda i: (0, i, 0)),
            pl.BlockSpec((blk, HID), lambda i: (i, 0)),
            pl.BlockSpec((1, HID), lambda i: (0, 0)),
            pl.BlockSpec((HEADS, HID), lambda i: (0, 0)),
        ],
        out_specs=pl.BlockSpec((blk, HID), lambda i: (i, 0)),
        out_shape=jax.ShapeDtypeStruct((n, HID), jnp.float32),
    )(acc64, denp, h, b2d, e4)


_B1 = 1000
_EPW = E_TOT // NW
_NB1 = _EPW // _B1
_NPAD = 51200
_RPW = _NPAD // NS

_mesh = plsc.VectorSubcoreMesh(core_axis_name="c", subcore_axis_name="s",
                               num_cores=NC, num_subcores=NS)


@functools.partial(
    pl.kernel, mesh=_mesh,
    compiler_params=pltpu.CompilerParams(use_tc_tiling_on_sc=False, needs_layout_passes=False),
    out_type=[
        jax.ShapeDtypeStruct((E_TOT, 16), jnp.float32),
        jax.ShapeDtypeStruct((NC * _NPAD, 16), jnp.float32),
    ],
    scratch_types=[
        pltpu.VMEM((_B1,), jnp.int32),
        pltpu.VMEM((_B1,), jnp.int32),
        pltpu.VMEM((_B1, 16), jnp.float32),
        pltpu.VMEM((_B1, 16), jnp.float32),
        pltpu.VMEM((_B1, 16), jnp.float32),
        pltpu.VMEM_SHARED((_NPAD, 16), jnp.float32),
        pltpu.SemaphoreType.DMA,
        pltpu.SemaphoreType.DMA,
    ],
)
def _p1(src_hbm, dst_hbm, sds_hbm, sdd_hbm, z16_hbm, ex_hbm, denp_hbm,
        idx_s, idx_d, srow, drow, exb, den_sh, sem1, sem2):
    c = lax.axis_index("c")
    s = lax.axis_index("s")
    wid = c * NS + s
    pltpu.sync_copy(z16_hbm.at[pl.ds(s * _RPW, _RPW)],
                    den_sh.at[pl.ds(s * _RPW, _RPW)])
    plsc.subcore_barrier()
    lane = lax.broadcasted_iota(jnp.int32, (LANES,), 0)
    m4 = lane < HEADS
    base = wid * _EPW

    def blk(b, _):
        off = base + b * _B1
        pltpu.sync_copy(src_hbm.at[pl.ds(off, _B1)], idx_s)
        pltpu.sync_copy(dst_hbm.at[pl.ds(off, _B1)], idx_d)
        d1 = pltpu.async_copy(sds_hbm.at[idx_s], srow, sem1)
        d2 = pltpu.async_copy(sdd_hbm.at[idx_d], drow, sem2)
        d1.wait()
        d2.wait()

        def edge(e, _):
            t = srow[e] + drow[e]
            l = jnp.maximum(t, 0.2 * t)
            exb[e] = jnp.where(m4, jnp.exp(l), 0.0)
            return _

        lax.fori_loop(0, _B1, edge, None)
        pltpu.sync_copy(exb, ex_hbm.at[pl.ds(off, _B1)])
        pltpu.sync_copy(exb, den_sh.at[idx_d], add=True)
        return _

    lax.fori_loop(0, _NB1, blk, None)
    plsc.subcore_barrier()
    pltpu.sync_copy(den_sh.at[pl.ds(s * _RPW, _RPW)],
                    denp_hbm.at[pl.ds(c * _NPAD + s * _RPW, _RPW)])


_B2 = 400
_EPT = E_TOT // NS
_NB2 = _EPT // _B2


@functools.partial(
    pl.kernel, mesh=_mesh,
    compiler_params=pltpu.CompilerParams(use_tc_tiling_on_sc=False, needs_layout_passes=False),
    out_type=jax.ShapeDtypeStruct((NC * _NPAD, 2, 16), jnp.float32),
    scratch_types=[
        pltpu.VMEM((_B2,), jnp.int32),
        pltpu.VMEM((_B2,), jnp.int32),
        pltpu.VMEM((_B2, 16), jnp.float32),
        pltpu.VMEM((_B2, 2, 16), jnp.float32),
        pltpu.VMEM_SHARED((_NPAD, 2, 16), jnp.float32),
        pltpu.SemaphoreType.DMA,
    ],
)
def _p2(src_hbm, dst_hbm, ex_hbm, xwcat_hbm, z32_hbm, accp_hbm,
        idx_s, idx_d, exv, xwrow, acc_sh, sem1):
    c = lax.axis_index("c")
    s = lax.axis_index("s")
    pltpu.sync_copy(z32_hbm.at[pl.ds(s * _RPW, _RPW)],
                    acc_sh.at[pl.ds(s * _RPW, _RPW)])
    plsc.subcore_barrier()
    off_c = c * _NPAD
    col0 = jnp.full((LANES,), 2 * c, jnp.int32)
    col1 = col0 + 1
    base = s * _EPT

    def blk(b, _):
        off = base + b * _B2
        pltpu.sync_copy(src_hbm.at[pl.ds(off, _B2)], idx_s)
        pltpu.sync_copy(dst_hbm.at[pl.ds(off, _B2)], idx_d)
        pltpu.sync_copy(ex_hbm.at[pl.ds(off, _B2)], exv)

        def addoff(k, _):
            sl = pl.ds(k * LANES, LANES)
            idx_s[sl] = idx_s[sl] + off_c
            return _

        lax.fori_loop(0, _B2 // LANES, addoff, None)
        pltpu.async_copy(xwcat_hbm.at[idx_s], xwrow, sem1).wait()

        def edge(e, _):
            row = jnp.full((LANES,), e, jnp.int32)
            m0 = plsc.load_gather(exv, [row, col0])
            xwrow[e, 0] = xwrow[e, 0] * m0
            m1 = plsc.load_gather(exv, [row, col1])
            xwrow[e, 1] = xwrow[e, 1] * m1
            return _

        lax.fori_loop(0, _B2, edge, None)
        pltpu.sync_copy(xwrow, acc_sh.at[idx_d], add=True)
        return _

    lax.fori_loop(0, _NB2, blk, None)
    plsc.subcore_barrier()
    pltpu.sync_copy(acc_sh.at[pl.ds(s * _RPW, _RPW)],
                    accp_hbm.at[pl.ds(off_c + s * _RPW, _RPW)])


def _to_xwcat(xw):
    xwp = jnp.pad(xw.reshape(N_TOT, 2, 32), ((0, _NPAD - N_TOT), (0, 0), (0, 0)))
    return xwp.transpose(1, 0, 2).reshape(NC * _NPAD, 2, 16)


def _from_accp(accp):
    return (accp.reshape(NC, _NPAD, 32)[:, :N_TOT]
            .transpose(1, 0, 2).reshape(N_TOT, HID))


def kernel(agent_hist, lane_nodes, edge_index_aa, edge_index_al,
           W_ih_a, W_hh_a, b_ih_a, b_hh_a,
           W_ih_l, W_hh_l, b_ih_l, b_hh_l,
           W1, as1, ad1, b1, bn_g, bn_b, bn_m, bn_v,
           W2, as2, ad2, b2):
    g16 = jnp.asarray(_g16)
    e4 = jnp.asarray(_e4)

    ah_t = jnp.transpose(agent_hist, (1, 0, 2))
    ln_t = jnp.transpose(lane_nodes, (1, 0, 2))
    agent_emb = _lstm_encode(ah_t, W_ih_a.T, W_hh_a.T,
                             (b_ih_a + b_hh_a).reshape(1, -1),
                             T_H, AG_IN, False, 400)
    lane_emb = _lstm_encode(ln_t, W_ih_l.T, W_hh_l.T,
                            (b_ih_l + b_hh_l).reshape(1, -1),
                            P_LEN, LN_IN, True, 800)
    x = jnp.concatenate([agent_emb, lane_emb], axis=0)

    ag = edge_index_al[0]
    ln = edge_index_al[1] + A_TOT
    src = jnp.concatenate([edge_index_aa[0], ag, ln])
    dst = jnp.concatenate([edge_index_aa[1], ln, ag])

    z16 = jnp.zeros((_NPAD, 16), jnp.float32)
    z32 = jnp.zeros((_NPAD, 2, 16), jnp.float32)

    xw1, sds1, sdd1 = _prep(x, W1.T, as1.reshape(1, -1), ad1.reshape(1, -1), g16)
    xwcat1 = _to_xwcat(xw1)
    ex1, denp1 = _p1(src, dst, sds1, sdd1, z16)
    accp1 = _p2(src, dst, ex1, xwcat1, z32)
    acc64_1 = _from_accp(accp1)
    denp1r = denp1.reshape(NC, _NPAD, 16)[:, :N_TOT]

    scale = bn_g / jnp.sqrt(bn_v + 1e-5)
    cst = (b1 - bn_m) * scale + bn_b

    h, xw2, sds2, sdd2 = _mid(acc64_1, denp1r, x, scale.reshape(1, -1),
                              cst.reshape(1, -1), e4, W2.T,
                              as2.reshape(1, -1), ad2.reshape(1, -1), g16)

    xwcat2 = _to_xwcat(xw2)
    ex2, denp2 = _p1(src, dst, sds2, sdd2, z16)
    accp2 = _p2(src, dst, ex2, xwcat2, z32)
    acc64_2 = _from_accp(accp2)
    denp2r = denp2.reshape(NC, _NPAD, 16)[:, :N_TOT]

    out = _final(acc64_2, denp2r, h, b2.reshape(1, -1), e4)

    return agent_emb, out[:A_TOT], lane_emb, out[A_TOT:]
